# baseline (speedup 1.0000x reference)
import functools

import jax
import jax.numpy as jnp
from jax import lax
from jax.experimental import pallas as pl
from jax.experimental.pallas import tpu as pltpu
from jax.experimental.pallas import tpu_sc as plsc

_B = 16384
_D = 64
_N = 100000
_NC = 2
_NS = 16
_NW = _NC * _NS
_BPW = _B // _NW
_PAD = 128
_PB = 16384
_PGRID = -(-_N // _PB)


def _p_body(idx_ref, *refs):
    eut_ref = refs[_D]
    p_ref = refs[_D + 1]
    h_ref = refs[_D + 2]
    g = pl.program_id(0)

    @pl.when(g == 0)
    def _build_h():
        acc = jnp.zeros((_D, _D), jnp.float32)
        for k in range(_D):
            lane = idx_ref[k] % _PAD
            onehot = (lax.broadcasted_iota(jnp.int32, (_PAD, _D), 0) == lane)
            col = jnp.dot(refs[k][...], onehot.astype(jnp.float32),
                          preferred_element_type=jnp.float32)
            sel = lax.broadcasted_iota(jnp.int32, (_D, _D), 1) == k
            acc = jnp.where(sel, col, acc)
        h_ref[...] = acc

    @pl.when(g > 0)
    def _matmul():
        p = lax.dot_general(
            eut_ref[...], h_ref[...], (((0,), (1,)), ((), ())),
            preferred_element_type=jnp.float32,
        )
        p_ref[:, :_D] = p


_p_fused = pl.pallas_call(
    _p_body,
    grid_spec=pltpu.PrefetchScalarGridSpec(
        num_scalar_prefetch=1,
        grid=(1 + _PGRID,),
        in_specs=[
            pl.BlockSpec((_D, _PAD),
                         functools.partial(
                             lambda k, g, idx: (0, idx[k] // _PAD), k))
            for k in range(_D)
        ] + [
            pl.BlockSpec((_D, _PB),
                         lambda g, idx: (0, jnp.maximum(g - 1, 0))),
        ],
        out_specs=pl.BlockSpec((_PB, _PAD),
                               lambda g, idx: (jnp.maximum(g - 1, 0), 0)),
        scratch_shapes=[pltpu.VMEM((_D, _D), jnp.float32)],
    ),
    out_shape=jax.ShapeDtypeStruct((_N, _PAD), jnp.float32),
)


@functools.partial(
    pl.kernel,
    out_type=jax.ShapeDtypeStruct((_B, _PAD), jnp.float32),
    mesh=plsc.VectorSubcoreMesh(core_axis_name="c", subcore_axis_name="s"),
    scratch_types=[
        pltpu.VMEM((_BPW,), jnp.int32),
        pltpu.VMEM((_BPW, _PAD), jnp.float32),
        pltpu.SemaphoreType.DMA,
    ],
)
def _sc_gather(user_idx_hbm, p_hbm, out_hbm, idx_v, rows_v, sem):
    wid = lax.axis_index("s") * _NC + lax.axis_index("c")
    base = wid * _BPW
    pltpu.sync_copy(user_idx_hbm.at[pl.ds(base, _BPW)], idx_v)
    pltpu.async_copy(p_hbm.at[idx_v], rows_v, sem).wait()
    pltpu.sync_copy(rows_v, out_hbm.at[pl.ds(base, _BPW)])


def kernel(user_idx, item_idx, embed_user, embed_item):
    user_idx = user_idx.astype(jnp.int32)
    item_idx = item_idx.astype(jnp.int32)
    eu_t = embed_user.T
    ei_t = embed_item.T
    p = _p_fused(item_idx, *([ei_t] * _D), eu_t)
    outp = _sc_gather(user_idx, p)
    return outp[:, :_D]


# speedup vs baseline: 1.1254x; 1.1254x over previous
import functools

import jax
import jax.numpy as jnp
from jax import lax
from jax.experimental import pallas as pl
from jax.experimental.pallas import tpu as pltpu
from jax.experimental.pallas import tpu_sc as plsc
import jax.experimental.layout
from jax.experimental.layout import Format, Layout

_B = 16384
_D = 64
_N = 100000
_NC = 2
_NS = 16
_NW = _NC * _NS
_BPW = _B // _NW
_PAD = 128
_PB = 16384
_PGRID = -(-_N // _PB)


def _p_body(idx_ref, *refs):
    eut_ref = refs[_D]
    p_ref = refs[_D + 1]
    h_ref = refs[_D + 2]
    g = pl.program_id(0)

    @pl.when(g == 0)
    def _build_h():
        acc = jnp.zeros((_D, _D), jnp.float32)
        for k in range(_D):
            lane = idx_ref[k] % _PAD
            onehot = (lax.broadcasted_iota(jnp.int32, (_PAD, _D), 0) == lane)
            col = jnp.dot(refs[k][...], onehot.astype(jnp.float32),
                          preferred_element_type=jnp.float32)
            sel = lax.broadcasted_iota(jnp.int32, (_D, _D), 1) == k
            acc = jnp.where(sel, col, acc)
        h_ref[...] = acc

    @pl.when(g > 0)
    def _matmul():
        p = lax.dot_general(
            eut_ref[...], h_ref[...], (((0,), (1,)), ((), ())),
            preferred_element_type=jnp.float32,
        )
        p_ref[:, :_D] = p


_p_fused = pl.pallas_call(
    _p_body,
    grid_spec=pltpu.PrefetchScalarGridSpec(
        num_scalar_prefetch=1,
        grid=(1 + _PGRID,),
        in_specs=[
            pl.BlockSpec((_D, _PAD),
                         functools.partial(
                             lambda k, g, idx: (0, idx[k] // _PAD), k))
            for k in range(_D)
        ] + [
            pl.BlockSpec((_D, _PB),
                         lambda g, idx: (0, jnp.maximum(g - 1, 0))),
        ],
        out_specs=pl.BlockSpec((_PB, _PAD),
                               lambda g, idx: (jnp.maximum(g - 1, 0), 0)),
        scratch_shapes=[pltpu.VMEM((_D, _D), jnp.float32)],
    ),
    out_shape=jax.ShapeDtypeStruct((_N, _PAD), jnp.float32),
)


@functools.partial(
    pl.kernel,
    out_type=jax.ShapeDtypeStruct((_B, _PAD), jnp.float32),
    mesh=plsc.VectorSubcoreMesh(core_axis_name="c", subcore_axis_name="s"),
    scratch_types=[
        pltpu.VMEM((_BPW,), jnp.int32),
        pltpu.VMEM((_BPW, _PAD), jnp.float32),
        pltpu.SemaphoreType.DMA,
    ],
)
def _sc_gather(user_idx_hbm, p_hbm, out_hbm, idx_v, rows_v, sem):
    wid = lax.axis_index("s") * _NC + lax.axis_index("c")
    base = wid * _BPW
    pltpu.sync_copy(user_idx_hbm.at[pl.ds(base, _BPW)], idx_v)
    pltpu.async_copy(p_hbm.at[idx_v], rows_v, sem).wait()
    pltpu.sync_copy(rows_v, out_hbm.at[pl.ds(base, _BPW)])


def kernel(user_idx, item_idx, embed_user, embed_item):
    user_idx = user_idx.astype(jnp.int32)
    item_idx = item_idx.astype(jnp.int32)
    eu_t = embed_user.T
    ei_t = embed_item.T
    p = _p_fused(item_idx, *([ei_t] * _D), eu_t)
    outp = _sc_gather(user_idx, p)
    out = outp[:, :_D]
    return jax.experimental.layout.with_layout_constraint(
        out, Layout((0, 1), ((8, 128),)))
